# baseline (speedup 1.0000x reference)
import functools
import numpy as np
import jax
import jax.numpy as jnp
from jax import lax
from jax.experimental import pallas as pl
from jax.experimental.pallas import tpu as pltpu
from jax.experimental.pallas import tpu_sc as plsc

_N_NODES = 10000
_BLK = 1280


@functools.lru_cache(maxsize=2)
def _fixed_perm(n: int):
    try:
        with jax.ensure_compile_time_eval():
            return np.asarray(jax.random.permutation(jax.random.key(42), n))
    except Exception:
        return None


def _wave(copies_iter, wave=16):
    pending = []
    for c in copies_iter:
        pending.append(c)
        if len(pending) == wave:
            for p in pending:
                p.wait()
            pending = []
    for p in pending:
        p.wait()


@functools.lru_cache(maxsize=2)
def _make_sampling_kernel(E: int):
    rows = 1280
    half = rows // 2
    perq = half // 16
    perp = rows // 16
    tsize = _N_NODES * _N_NODES + 16384

    mesh = plsc.VectorSubcoreMesh(core_axis_name="c", subcore_axis_name="s")

    @functools.partial(
        pl.kernel,
        mesh=mesh,
        out_type=[
            jax.ShapeDtypeStruct((2 * tsize,), jnp.int32),
            jax.ShapeDtypeStruct((rows, 128), jnp.int32),
            jax.ShapeDtypeStruct((rows, 128), jnp.float32),
        ],
        scratch_types=[
            pltpu.VMEM((perq, 128), jnp.int32),
            pltpu.VMEM((perq, 128), jnp.int32),
            pltpu.VMEM((perq, 128), jnp.int32),
            pltpu.VMEM((perq, 128), jnp.int32),
            pltpu.VMEM((perq, 128), jnp.float32),
            pltpu.VMEM((perp, 128), jnp.int32),
            pltpu.VMEM((perp, 128), jnp.int32),
            pltpu.VMEM((128,), jnp.int32),
            pltpu.VMEM((128,), jnp.int32),
            pltpu.SemaphoreType.DMA,
            pltpu.SemaphoreType.DMA,
        ],
    )
    def sampling(src2_h, dst2_h, dstf_h, perm2_h, table_h, dperm_h, kf_h,
                 gat_v, src_v, hneg_v, pres_v, kf_v, pa_v, pb_v, z_v, one_v,
                 sem_g, sem_s):
        c = lax.axis_index("c")
        w = lax.axis_index("s")
        toff = c * tsize
        qrow0 = c * half + w * perq
        prow0 = w * perp

        @pl.loop(0, 8)
        def _(k):
            z_v[pl.ds(k * 16, 16)] = jnp.zeros((16,), jnp.int32)
            one_v[pl.ds(k * 16, 16)] = jnp.full((16,), 1, jnp.int32)

        pltpu.sync_copy(perm2_h.at[pl.ds(qrow0, perq)], pres_v)
        pltpu.sync_copy(src2_h.at[pl.ds(qrow0, perq)], src_v)
        _wave(pltpu.async_copy(dstf_h.at[pres_v.at[j]], gat_v.at[j], sem_g)
              for j in range(perq))

        @pl.loop(0, perq)
        def _(j):
            for k in range(8):
                sl = pl.ds(k * 16, 16)
                hneg_v[j, sl] = src_v[j, sl] * _N_NODES + gat_v[j, sl] + toff

        pltpu.sync_copy(gat_v, dperm_h.at[pl.ds(qrow0, perq)])
        _wave(pltpu.async_copy(z_v, table_h.at[hneg_v.at[j]], sem_s)
              for j in range(perq))
        plsc.subcore_barrier()

        pltpu.sync_copy(src2_h.at[pl.ds(prow0, perp)], pa_v)
        pltpu.sync_copy(dst2_h.at[pl.ds(prow0, perp)], pb_v)

        @pl.loop(0, perp)
        def _(j):
            for k in range(8):
                sl = pl.ds(k * 16, 16)
                pb_v[j, sl] = pa_v[j, sl] * _N_NODES + pb_v[j, sl] + toff

        _wave(pltpu.async_copy(one_v, table_h.at[pb_v.at[j]], sem_s)
              for j in range(perp))
        plsc.subcore_barrier()

        _wave(pltpu.async_copy(table_h.at[hneg_v.at[j]], pres_v.at[j], sem_g)
              for j in range(perq))

        @pl.loop(0, perq)
        def _(j):
            for k in range(8):
                sl = pl.ds(k * 16, 16)
                ok = (src_v[j, sl] != gat_v[j, sl]) & (pres_v[j, sl] == 0)
                kf_v[j, sl] = jnp.where(ok, 1.0, 0.0).astype(jnp.float32)

        pltpu.sync_copy(kf_v, kf_h.at[pl.ds(qrow0, perq)])

    return sampling


@functools.lru_cache(maxsize=2)
def _make_pair_gather_kernel(E: int):
    crows = E // 32
    p = crows // 32
    rem = crows % 32
    n4 = p + 2

    mesh = plsc.VectorSubcoreMesh(core_axis_name="c", subcore_axis_name="s")

    @functools.partial(
        pl.kernel,
        mesh=mesh,
        out_type=jax.ShapeDtypeStruct((E, 256), jnp.float32),
        scratch_types=[
            pltpu.VMEM((32,), jnp.int32),
            pltpu.VMEM((32,), jnp.int32),
            pltpu.VMEM((32,), jnp.int32),
            pltpu.VMEM((32,), jnp.int32),
            pltpu.VMEM((32, 256), jnp.float32),
            pltpu.VMEM((32, 256), jnp.float32),
            pltpu.VMEM((32, 256), jnp.float32),
            pltpu.VMEM((32, 256), jnp.float32),
            pltpu.SemaphoreType.DMA,
            pltpu.SemaphoreType.DMA,
        ],
    )
    def pair_gather(a_h, b_h, srcc_h, dpc_h, g_h,
                    ia0, ia1, ib0, ib1, ga0, ga1, gb0, gb1, sem0, sem1):
        w = lax.axis_index("s") * 2 + lax.axis_index("c")
        r0 = jnp.minimum(w * p + jnp.minimum(w, rem), crows - n4)
        ia = (ia0, ia1)
        ib = (ib0, ib1)
        ga = (ga0, ga1)
        gb = (gb0, gb1)
        sems = (sem0, sem1)

        def issue(chunk, b):
            pltpu.sync_copy(srcc_h.at[pl.ds(chunk * 32, 32)], ia[b])
            pltpu.sync_copy(dpc_h.at[pl.ds(chunk * 32, 32)], ib[b])
            pltpu.async_copy(a_h.at[ia[b]], ga[b], sems[b])
            pltpu.async_copy(b_h.at[ib[b]], gb[b], sems[b])

        issue(r0, 0)
        issue(r0 + 1, 1)

        @pl.loop(0, n4, step=2)
        def _(i):
            for b in range(2):
                chunk = r0 + i + b
                pltpu.make_async_copy(a_h.at[ia[b]], ga[b], sems[b]).wait()
                pltpu.make_async_copy(b_h.at[ib[b]], gb[b], sems[b]).wait()

                @pl.loop(0, 32)
                def _(r):
                    for k in range(16):
                        sl = pl.ds(k * 16, 16)
                        ga[b][r, sl] = ga[b][r, sl] + gb[b][r, sl]

                pltpu.sync_copy(ga[b], g_h.at[pl.ds(chunk * 32, 32)])

                @pl.when(i + b + 2 < n4)
                def _():
                    issue(chunk + 2, b)

    return pair_gather


def _loss_kernel(hs_ref, hd_ref, g_ref, keep_ref, W1_ref, w2_ref, bias_ref,
                 pos_ref, neg_ref, cnt_ref):
    W1 = W1_ref[...]
    w1a = W1[:256].astype(jnp.bfloat16)
    w1b = W1[256:].astype(jnp.bfloat16)
    hs = hs_ref[...].astype(jnp.bfloat16)
    hd = hd_ref[...].astype(jnp.bfloat16)
    b1 = bias_ref[0:1, :]
    b2 = bias_ref[1:2, 0:1]
    w2 = w2_ref[...]

    dn = (((1,), (0,)), ((), ()))
    pre = (lax.dot_general(hs, w1a, dn, preferred_element_type=jnp.float32)
           + lax.dot_general(hd, w1b, dn, preferred_element_type=jnp.float32)
           + b1)
    hpos = jnp.maximum(pre, 0.0)
    pos_score = jnp.sum(hpos * w2, axis=1, keepdims=True) + b2
    pos_ls = jax.nn.log_sigmoid(pos_score)

    hneg = jnp.maximum(g_ref[...] + b1, 0.0)
    neg_score = jnp.sum(hneg * w2, axis=1, keepdims=True) + b2
    keepf = keep_ref[...]
    neg_ls = jax.nn.log_sigmoid(-neg_score) * keepf

    @pl.when(pl.program_id(0) == 0)
    def _():
        pos_ref[...] = jnp.zeros_like(pos_ref)
        neg_ref[...] = jnp.zeros_like(neg_ref)
        cnt_ref[...] = jnp.zeros_like(cnt_ref)

    pos_ref[...] += jnp.sum(pos_ls, axis=0, keepdims=True).sum(axis=1, keepdims=True)
    neg_ref[...] += jnp.sum(neg_ls, axis=0, keepdims=True).sum(axis=1, keepdims=True)
    cnt_ref[...] += jnp.sum(keepf, axis=0, keepdims=True).sum(axis=1, keepdims=True)


def _fused_loss(h_src, h_dst, G, keepf, W1, w2row, bias):
    E = h_src.shape[0]
    grid = (E // _BLK,)
    acc = jax.ShapeDtypeStruct((1, 1), jnp.float32)
    row_spec = pl.BlockSpec((_BLK, 256), lambda i: (i, 0))
    return pl.pallas_call(
        _loss_kernel,
        grid=grid,
        in_specs=[
            row_spec, row_spec, row_spec,
            pl.BlockSpec((_BLK, 1), lambda i: (i, 0)),
            pl.BlockSpec((512, 256), lambda i: (0, 0)),
            pl.BlockSpec((1, 256), lambda i: (0, 0)),
            pl.BlockSpec((2, 256), lambda i: (0, 0)),
        ],
        out_specs=[
            pl.BlockSpec((1, 1), lambda i: (0, 0)),
            pl.BlockSpec((1, 1), lambda i: (0, 0)),
            pl.BlockSpec((1, 1), lambda i: (0, 0)),
        ],
        out_shape=[acc, acc, acc],
        compiler_params=pltpu.CompilerParams(
            dimension_semantics=("arbitrary",)),
    )(h_src, h_dst, G, keepf, W1, w2row, bias)


def kernel(h_src, h_dst, edge_index, inference, W1, b1, W2, b2):
    E, D = h_src.shape
    src = edge_index[0]
    dst = edge_index[1]
    perm_np = _fixed_perm(E)
    if perm_np is not None:
        perm = jnp.asarray(perm_np)
    else:
        perm = jax.random.permutation(jax.random.key(42), E)

    rows = E // 128
    pad = 1280 - rows
    src2 = jnp.concatenate(
        [src.reshape(rows, 128),
         jnp.full((pad, 128), _N_NODES, jnp.int32)], axis=0)
    dst2 = jnp.concatenate(
        [dst.reshape(rows, 128), jnp.zeros((pad, 128), jnp.int32)], axis=0)
    perm2 = jnp.concatenate(
        [perm.reshape(rows, 128).astype(jnp.int32),
         jnp.zeros((pad, 128), jnp.int32)], axis=0)

    _, dperm2, kf2 = _make_sampling_kernel(E)(src2, dst2, dst, perm2)
    dperm = dperm2[:rows].reshape(E)
    keepf = kf2[:rows].reshape(E, 1)

    e_iota = jnp.arange(E, dtype=jnp.int32)
    Lsrc = jnp.zeros((_N_NODES,), jnp.int32).at[src].max(e_iota)
    Ldst = jnp.zeros((_N_NODES,), jnp.int32).at[dst].max(e_iota)

    A = jnp.take(h_src, Lsrc, axis=0) @ W1[:D]
    B = jnp.take(h_dst, Ldst, axis=0) @ W1[D:]

    G = _make_pair_gather_kernel(E)(A, B, src, dperm)

    w2row = W2.reshape(1, D)
    bias = jnp.concatenate(
        [b1.reshape(1, D), jnp.broadcast_to(b2.reshape(1, 1), (1, D))], axis=0)

    pos_sum, neg_sum, keep_sum = _fused_loss(h_src, h_dst, G, keepf, W1, w2row, bias)
    return -(pos_sum[0, 0] / E + neg_sum[0, 0] / keep_sum[0, 0])


# speedup vs baseline: 1.0004x; 1.0004x over previous
import functools
import numpy as np
import jax
import jax.numpy as jnp
from jax import lax
from jax.experimental import pallas as pl
from jax.experimental.pallas import tpu as pltpu
from jax.experimental.pallas import tpu_sc as plsc

_N_NODES = 10000
_BLK = 1280


@functools.lru_cache(maxsize=2)
def _fixed_perm(n: int):
    try:
        with jax.ensure_compile_time_eval():
            return np.asarray(jax.random.permutation(jax.random.key(42), n))
    except Exception:
        return None


def _wave(copies_iter, wave=64):
    pending = []
    for c in copies_iter:
        pending.append(c)
        if len(pending) == wave:
            for p in pending:
                p.wait()
            pending = []
    for p in pending:
        p.wait()


@functools.lru_cache(maxsize=2)
def _make_sampling_kernel(E: int):
    rows = 1280
    half = rows // 2
    perq = half // 16
    perp = rows // 16
    tsize = _N_NODES * _N_NODES + 16384

    mesh = plsc.VectorSubcoreMesh(core_axis_name="c", subcore_axis_name="s")

    @functools.partial(
        pl.kernel,
        mesh=mesh,
        out_type=[
            jax.ShapeDtypeStruct((2 * tsize,), jnp.int32),
            jax.ShapeDtypeStruct((rows, 128), jnp.int32),
            jax.ShapeDtypeStruct((rows, 128), jnp.float32),
        ],
        scratch_types=[
            pltpu.VMEM((perq, 128), jnp.int32),
            pltpu.VMEM((perq, 128), jnp.int32),
            pltpu.VMEM((perq, 128), jnp.int32),
            pltpu.VMEM((perq, 128), jnp.int32),
            pltpu.VMEM((perq, 128), jnp.float32),
            pltpu.VMEM((perp, 128), jnp.int32),
            pltpu.VMEM((perp, 128), jnp.int32),
            pltpu.VMEM((128,), jnp.int32),
            pltpu.VMEM((128,), jnp.int32),
            pltpu.SemaphoreType.DMA,
            pltpu.SemaphoreType.DMA,
        ],
    )
    def sampling(src2_h, dst2_h, dstf_h, perm2_h, table_h, dperm_h, kf_h,
                 gat_v, src_v, hneg_v, pres_v, kf_v, pa_v, pb_v, z_v, one_v,
                 sem_g, sem_s):
        c = lax.axis_index("c")
        w = lax.axis_index("s")
        toff = c * tsize
        qrow0 = c * half + w * perq
        prow0 = w * perp

        @pl.loop(0, 8)
        def _(k):
            z_v[pl.ds(k * 16, 16)] = jnp.zeros((16,), jnp.int32)
            one_v[pl.ds(k * 16, 16)] = jnp.full((16,), 1, jnp.int32)

        pltpu.sync_copy(perm2_h.at[pl.ds(qrow0, perq)], pres_v)
        pltpu.sync_copy(src2_h.at[pl.ds(qrow0, perq)], src_v)
        _wave(pltpu.async_copy(dstf_h.at[pres_v.at[j]], gat_v.at[j], sem_g)
              for j in range(perq))

        @pl.loop(0, perq)
        def _(j):
            for k in range(8):
                sl = pl.ds(k * 16, 16)
                hneg_v[j, sl] = src_v[j, sl] * _N_NODES + gat_v[j, sl] + toff

        pltpu.sync_copy(gat_v, dperm_h.at[pl.ds(qrow0, perq)])
        _wave(pltpu.async_copy(z_v, table_h.at[hneg_v.at[j]], sem_s)
              for j in range(perq))
        plsc.subcore_barrier()

        pltpu.sync_copy(src2_h.at[pl.ds(prow0, perp)], pa_v)
        pltpu.sync_copy(dst2_h.at[pl.ds(prow0, perp)], pb_v)

        @pl.loop(0, perp)
        def _(j):
            for k in range(8):
                sl = pl.ds(k * 16, 16)
                pb_v[j, sl] = pa_v[j, sl] * _N_NODES + pb_v[j, sl] + toff

        _wave(pltpu.async_copy(one_v, table_h.at[pb_v.at[j]], sem_s)
              for j in range(perp))
        plsc.subcore_barrier()

        _wave(pltpu.async_copy(table_h.at[hneg_v.at[j]], pres_v.at[j], sem_g)
              for j in range(perq))

        @pl.loop(0, perq)
        def _(j):
            for k in range(8):
                sl = pl.ds(k * 16, 16)
                ok = (src_v[j, sl] != gat_v[j, sl]) & (pres_v[j, sl] == 0)
                kf_v[j, sl] = jnp.where(ok, 1.0, 0.0).astype(jnp.float32)

        pltpu.sync_copy(kf_v, kf_h.at[pl.ds(qrow0, perq)])

    return sampling


@functools.lru_cache(maxsize=2)
def _make_pair_gather_kernel(E: int):
    crows = E // 32
    p = crows // 32
    rem = crows % 32
    n4 = p + 2

    mesh = plsc.VectorSubcoreMesh(core_axis_name="c", subcore_axis_name="s")

    @functools.partial(
        pl.kernel,
        mesh=mesh,
        out_type=jax.ShapeDtypeStruct((E, 256), jnp.float32),
        scratch_types=[
            pltpu.VMEM((32,), jnp.int32),
            pltpu.VMEM((32,), jnp.int32),
            pltpu.VMEM((32,), jnp.int32),
            pltpu.VMEM((32,), jnp.int32),
            pltpu.VMEM((32, 256), jnp.float32),
            pltpu.VMEM((32, 256), jnp.float32),
            pltpu.VMEM((32, 256), jnp.float32),
            pltpu.VMEM((32, 256), jnp.float32),
            pltpu.SemaphoreType.DMA,
            pltpu.SemaphoreType.DMA,
        ],
    )
    def pair_gather(a_h, b_h, srcc_h, dpc_h, g_h,
                    ia0, ia1, ib0, ib1, ga0, ga1, gb0, gb1, sem0, sem1):
        w = lax.axis_index("s") * 2 + lax.axis_index("c")
        r0 = jnp.minimum(w * p + jnp.minimum(w, rem), crows - n4)
        ia = (ia0, ia1)
        ib = (ib0, ib1)
        ga = (ga0, ga1)
        gb = (gb0, gb1)
        sems = (sem0, sem1)

        def issue(chunk, b):
            pltpu.sync_copy(srcc_h.at[pl.ds(chunk * 32, 32)], ia[b])
            pltpu.sync_copy(dpc_h.at[pl.ds(chunk * 32, 32)], ib[b])
            pltpu.async_copy(a_h.at[ia[b]], ga[b], sems[b])
            pltpu.async_copy(b_h.at[ib[b]], gb[b], sems[b])

        issue(r0, 0)
        issue(r0 + 1, 1)

        @pl.loop(0, n4, step=2)
        def _(i):
            for b in range(2):
                chunk = r0 + i + b
                pltpu.make_async_copy(a_h.at[ia[b]], ga[b], sems[b]).wait()
                pltpu.make_async_copy(b_h.at[ib[b]], gb[b], sems[b]).wait()

                @pl.loop(0, 32)
                def _(r):
                    for k in range(16):
                        sl = pl.ds(k * 16, 16)
                        ga[b][r, sl] = ga[b][r, sl] + gb[b][r, sl]

                pltpu.sync_copy(ga[b], g_h.at[pl.ds(chunk * 32, 32)])

                @pl.when(i + b + 2 < n4)
                def _():
                    issue(chunk + 2, b)

    return pair_gather


def _loss_kernel(hs_ref, hd_ref, g_ref, keep_ref, W1_ref, w2_ref, bias_ref,
                 pos_ref, neg_ref, cnt_ref):
    W1 = W1_ref[...]
    w1a = W1[:256].astype(jnp.bfloat16)
    w1b = W1[256:].astype(jnp.bfloat16)
    hs = hs_ref[...].astype(jnp.bfloat16)
    hd = hd_ref[...].astype(jnp.bfloat16)
    b1 = bias_ref[0:1, :]
    b2 = bias_ref[1:2, 0:1]
    w2 = w2_ref[...]

    dn = (((1,), (0,)), ((), ()))
    pre = (lax.dot_general(hs, w1a, dn, preferred_element_type=jnp.float32)
           + lax.dot_general(hd, w1b, dn, preferred_element_type=jnp.float32)
           + b1)
    hpos = jnp.maximum(pre, 0.0)
    pos_score = jnp.sum(hpos * w2, axis=1, keepdims=True) + b2
    pos_ls = jax.nn.log_sigmoid(pos_score)

    hneg = jnp.maximum(g_ref[...] + b1, 0.0)
    neg_score = jnp.sum(hneg * w2, axis=1, keepdims=True) + b2
    keepf = keep_ref[...]
    neg_ls = jax.nn.log_sigmoid(-neg_score) * keepf

    @pl.when(pl.program_id(0) == 0)
    def _():
        pos_ref[...] = jnp.zeros_like(pos_ref)
        neg_ref[...] = jnp.zeros_like(neg_ref)
        cnt_ref[...] = jnp.zeros_like(cnt_ref)

    pos_ref[...] += jnp.sum(pos_ls, axis=0, keepdims=True).sum(axis=1, keepdims=True)
    neg_ref[...] += jnp.sum(neg_ls, axis=0, keepdims=True).sum(axis=1, keepdims=True)
    cnt_ref[...] += jnp.sum(keepf, axis=0, keepdims=True).sum(axis=1, keepdims=True)


def _fused_loss(h_src, h_dst, G, keepf, W1, w2row, bias):
    E = h_src.shape[0]
    grid = (E // _BLK,)
    acc = jax.ShapeDtypeStruct((1, 1), jnp.float32)
    row_spec = pl.BlockSpec((_BLK, 256), lambda i: (i, 0))
    return pl.pallas_call(
        _loss_kernel,
        grid=grid,
        in_specs=[
            row_spec, row_spec, row_spec,
            pl.BlockSpec((_BLK, 1), lambda i: (i, 0)),
            pl.BlockSpec((512, 256), lambda i: (0, 0)),
            pl.BlockSpec((1, 256), lambda i: (0, 0)),
            pl.BlockSpec((2, 256), lambda i: (0, 0)),
        ],
        out_specs=[
            pl.BlockSpec((1, 1), lambda i: (0, 0)),
            pl.BlockSpec((1, 1), lambda i: (0, 0)),
            pl.BlockSpec((1, 1), lambda i: (0, 0)),
        ],
        out_shape=[acc, acc, acc],
        compiler_params=pltpu.CompilerParams(
            dimension_semantics=("arbitrary",)),
    )(h_src, h_dst, G, keepf, W1, w2row, bias)


def kernel(h_src, h_dst, edge_index, inference, W1, b1, W2, b2):
    E, D = h_src.shape
    src = edge_index[0]
    dst = edge_index[1]
    perm_np = _fixed_perm(E)
    if perm_np is not None:
        perm = jnp.asarray(perm_np)
    else:
        perm = jax.random.permutation(jax.random.key(42), E)

    rows = E // 128
    pad = 1280 - rows
    src2 = jnp.concatenate(
        [src.reshape(rows, 128),
         jnp.full((pad, 128), _N_NODES, jnp.int32)], axis=0)
    dst2 = jnp.concatenate(
        [dst.reshape(rows, 128), jnp.zeros((pad, 128), jnp.int32)], axis=0)
    perm2 = jnp.concatenate(
        [perm.reshape(rows, 128).astype(jnp.int32),
         jnp.zeros((pad, 128), jnp.int32)], axis=0)

    _, dperm2, kf2 = _make_sampling_kernel(E)(src2, dst2, dst, perm2)
    dperm = dperm2[:rows].reshape(E)
    keepf = kf2[:rows].reshape(E, 1)

    e_iota = jnp.arange(E, dtype=jnp.int32)
    Lsrc = jnp.zeros((_N_NODES,), jnp.int32).at[src].max(e_iota)
    Ldst = jnp.zeros((_N_NODES,), jnp.int32).at[dst].max(e_iota)

    A = jnp.take(h_src, Lsrc, axis=0) @ W1[:D]
    B = jnp.take(h_dst, Ldst, axis=0) @ W1[D:]

    G = _make_pair_gather_kernel(E)(A, B, src, dperm)

    w2row = W2.reshape(1, D)
    bias = jnp.concatenate(
        [b1.reshape(1, D), jnp.broadcast_to(b2.reshape(1, 1), (1, D))], axis=0)

    pos_sum, neg_sum, keep_sum = _fused_loss(h_src, h_dst, G, keepf, W1, w2row, bias)
    return -(pos_sum[0, 0] / E + neg_sum[0, 0] / keep_sum[0, 0])


# speedup vs baseline: 1.0527x; 1.0523x over previous
import functools
import numpy as np
import jax
import jax.numpy as jnp
from jax import lax
from jax.experimental import pallas as pl
from jax.experimental.pallas import tpu as pltpu
from jax.experimental.pallas import tpu_sc as plsc

_N_NODES = 10000
_BLK = 1280


@functools.lru_cache(maxsize=2)
def _fixed_perm(n: int):
    try:
        with jax.ensure_compile_time_eval():
            return np.asarray(jax.random.permutation(jax.random.key(42), n))
    except Exception:
        return None


def _wave(copies_iter, wave=64):
    pending = []
    for c in copies_iter:
        pending.append(c)
        if len(pending) == wave:
            for p in pending:
                p.wait()
            pending = []
    for p in pending:
        p.wait()


@functools.lru_cache(maxsize=2)
def _make_sampling_kernel(E: int):
    rows = 1280
    per = rows // 16

    mesh = plsc.VectorSubcoreMesh(core_axis_name="c", subcore_axis_name="s")

    @functools.partial(
        pl.kernel,
        mesh=mesh,
        out_type=[
            jax.ShapeDtypeStruct((_N_NODES * _N_NODES + 16384,), jnp.int32),
            jax.ShapeDtypeStruct((rows, 128), jnp.int32),
            jax.ShapeDtypeStruct((rows, 128), jnp.float32),
        ],
        scratch_types=[
            pltpu.VMEM((per, 128), jnp.int32),
            pltpu.VMEM((per, 128), jnp.int32),
            pltpu.VMEM((per, 128), jnp.int32),
            pltpu.VMEM((per, 128), jnp.int32),
            pltpu.VMEM((per, 128), jnp.int32),
            pltpu.VMEM((per, 128), jnp.float32),
            pltpu.VMEM((128,), jnp.int32),
            pltpu.VMEM((128,), jnp.int32),
            pltpu.SemaphoreType.DMA,
            pltpu.SemaphoreType.DMA,
        ],
    )
    def sampling(src2_h, dst2_h, dstf_h, perm2_h, table_h, dperm_h, kf_h,
                 idx_v, gat_v, src_v, hneg_v, pres_v, kf_v, z_v, one_v,
                 sem_g, sem_s):
        core = lax.axis_index("c")

        @pl.when(core == 0)
        def _():
            w = lax.axis_index("s")
            row0 = w * per

            @pl.loop(0, 8)
            def _(k):
                z_v[pl.ds(k * 16, 16)] = jnp.zeros((16,), jnp.int32)
                one_v[pl.ds(k * 16, 16)] = jnp.full((16,), 1, jnp.int32)

            pltpu.sync_copy(perm2_h.at[pl.ds(row0, per)], idx_v)
            pltpu.sync_copy(src2_h.at[pl.ds(row0, per)], src_v)
            _wave(pltpu.async_copy(dstf_h.at[idx_v.at[j]], gat_v.at[j], sem_g)
                  for j in range(per))

            @pl.loop(0, per)
            def _(j):
                for k in range(8):
                    sl = pl.ds(k * 16, 16)
                    hneg_v[j, sl] = src_v[j, sl] * _N_NODES + gat_v[j, sl]

            pltpu.sync_copy(gat_v, dperm_h.at[pl.ds(row0, per)])
            _wave(pltpu.async_copy(z_v, table_h.at[hneg_v.at[j]], sem_s)
                  for j in range(per))
            plsc.subcore_barrier()

            pltpu.sync_copy(dst2_h.at[pl.ds(row0, per)], idx_v)

            @pl.loop(0, per)
            def _(j):
                for k in range(8):
                    sl = pl.ds(k * 16, 16)
                    idx_v[j, sl] = src_v[j, sl] * _N_NODES + idx_v[j, sl]

            _wave(pltpu.async_copy(one_v, table_h.at[idx_v.at[j]], sem_s)
                  for j in range(per))
            plsc.subcore_barrier()

            _wave(pltpu.async_copy(table_h.at[hneg_v.at[j]], pres_v.at[j], sem_g)
                  for j in range(per))

            @pl.loop(0, per)
            def _(j):
                for k in range(8):
                    sl = pl.ds(k * 16, 16)
                    ok = (src_v[j, sl] != gat_v[j, sl]) & (pres_v[j, sl] == 0)
                    kf_v[j, sl] = jnp.where(ok, 1.0, 0.0).astype(jnp.float32)

            pltpu.sync_copy(kf_v, kf_h.at[pl.ds(row0, per)])

    return sampling


@functools.lru_cache(maxsize=2)
def _make_pair_gather_kernel(E: int):
    crows = E // 32
    p = crows // 32
    rem = crows % 32
    n4 = p + 2

    mesh = plsc.VectorSubcoreMesh(core_axis_name="c", subcore_axis_name="s")

    @functools.partial(
        pl.kernel,
        mesh=mesh,
        out_type=jax.ShapeDtypeStruct((E, 256), jnp.float32),
        scratch_types=[
            pltpu.VMEM((32,), jnp.int32),
            pltpu.VMEM((32,), jnp.int32),
            pltpu.VMEM((32,), jnp.int32),
            pltpu.VMEM((32,), jnp.int32),
            pltpu.VMEM((32, 256), jnp.float32),
            pltpu.VMEM((32, 256), jnp.float32),
            pltpu.VMEM((32, 256), jnp.float32),
            pltpu.VMEM((32, 256), jnp.float32),
            pltpu.SemaphoreType.DMA,
            pltpu.SemaphoreType.DMA,
        ],
    )
    def pair_gather(a_h, b_h, srcc_h, dpc_h, g_h,
                    ia0, ia1, ib0, ib1, ga0, ga1, gb0, gb1, sem0, sem1):
        w = lax.axis_index("s") * 2 + lax.axis_index("c")
        r0 = jnp.minimum(w * p + jnp.minimum(w, rem), crows - n4)
        ia = (ia0, ia1)
        ib = (ib0, ib1)
        ga = (ga0, ga1)
        gb = (gb0, gb1)
        sems = (sem0, sem1)

        def issue(chunk, b):
            pltpu.sync_copy(srcc_h.at[pl.ds(chunk * 32, 32)], ia[b])
            pltpu.sync_copy(dpc_h.at[pl.ds(chunk * 32, 32)], ib[b])
            pltpu.async_copy(a_h.at[ia[b]], ga[b], sems[b])
            pltpu.async_copy(b_h.at[ib[b]], gb[b], sems[b])

        issue(r0, 0)
        issue(r0 + 1, 1)

        @pl.loop(0, n4, step=2)
        def _(i):
            for b in range(2):
                chunk = r0 + i + b
                pltpu.make_async_copy(a_h.at[ia[b]], ga[b], sems[b]).wait()
                pltpu.make_async_copy(b_h.at[ib[b]], gb[b], sems[b]).wait()

                @pl.loop(0, 32)
                def _(r):
                    for k in range(16):
                        sl = pl.ds(k * 16, 16)
                        ga[b][r, sl] = ga[b][r, sl] + gb[b][r, sl]

                pltpu.sync_copy(ga[b], g_h.at[pl.ds(chunk * 32, 32)])

                @pl.when(i + b + 2 < n4)
                def _():
                    issue(chunk + 2, b)

    return pair_gather


def _loss_kernel(hs_ref, hd_ref, g_ref, keep_ref, W1_ref, w2_ref, bias_ref,
                 pos_ref, neg_ref, cnt_ref):
    W1 = W1_ref[...]
    w1a = W1[:256].astype(jnp.bfloat16)
    w1b = W1[256:].astype(jnp.bfloat16)
    hs = hs_ref[...].astype(jnp.bfloat16)
    hd = hd_ref[...].astype(jnp.bfloat16)
    b1 = bias_ref[0:1, :]
    b2 = bias_ref[1:2, 0:1]
    w2 = w2_ref[...]

    dn = (((1,), (0,)), ((), ()))
    pre = (lax.dot_general(hs, w1a, dn, preferred_element_type=jnp.float32)
           + lax.dot_general(hd, w1b, dn, preferred_element_type=jnp.float32)
           + b1)
    hpos = jnp.maximum(pre, 0.0)
    pos_score = jnp.sum(hpos * w2, axis=1, keepdims=True) + b2
    pos_ls = jax.nn.log_sigmoid(pos_score)

    hneg = jnp.maximum(g_ref[...] + b1, 0.0)
    neg_score = jnp.sum(hneg * w2, axis=1, keepdims=True) + b2
    keepf = keep_ref[...]
    neg_ls = jax.nn.log_sigmoid(-neg_score) * keepf

    @pl.when(pl.program_id(0) == 0)
    def _():
        pos_ref[...] = jnp.zeros_like(pos_ref)
        neg_ref[...] = jnp.zeros_like(neg_ref)
        cnt_ref[...] = jnp.zeros_like(cnt_ref)

    pos_ref[...] += jnp.sum(pos_ls, axis=0, keepdims=True).sum(axis=1, keepdims=True)
    neg_ref[...] += jnp.sum(neg_ls, axis=0, keepdims=True).sum(axis=1, keepdims=True)
    cnt_ref[...] += jnp.sum(keepf, axis=0, keepdims=True).sum(axis=1, keepdims=True)


def _fused_loss(h_src, h_dst, G, keepf, W1, w2row, bias):
    E = h_src.shape[0]
    grid = (E // _BLK,)
    acc = jax.ShapeDtypeStruct((1, 1), jnp.float32)
    row_spec = pl.BlockSpec((_BLK, 256), lambda i: (i, 0))
    return pl.pallas_call(
        _loss_kernel,
        grid=grid,
        in_specs=[
            row_spec, row_spec, row_spec,
            pl.BlockSpec((_BLK, 1), lambda i: (i, 0)),
            pl.BlockSpec((512, 256), lambda i: (0, 0)),
            pl.BlockSpec((1, 256), lambda i: (0, 0)),
            pl.BlockSpec((2, 256), lambda i: (0, 0)),
        ],
        out_specs=[
            pl.BlockSpec((1, 1), lambda i: (0, 0)),
            pl.BlockSpec((1, 1), lambda i: (0, 0)),
            pl.BlockSpec((1, 1), lambda i: (0, 0)),
        ],
        out_shape=[acc, acc, acc],
        compiler_params=pltpu.CompilerParams(
            dimension_semantics=("arbitrary",)),
    )(h_src, h_dst, G, keepf, W1, w2row, bias)


def kernel(h_src, h_dst, edge_index, inference, W1, b1, W2, b2):
    E, D = h_src.shape
    src = edge_index[0]
    dst = edge_index[1]
    perm_np = _fixed_perm(E)
    if perm_np is not None:
        perm = jnp.asarray(perm_np)
    else:
        perm = jax.random.permutation(jax.random.key(42), E)

    rows = E // 128
    pad = 1280 - rows
    src2 = jnp.concatenate(
        [src.reshape(rows, 128),
         jnp.full((pad, 128), _N_NODES, jnp.int32)], axis=0)
    dst2 = jnp.concatenate(
        [dst.reshape(rows, 128), jnp.zeros((pad, 128), jnp.int32)], axis=0)
    perm2 = jnp.concatenate(
        [perm.reshape(rows, 128).astype(jnp.int32),
         jnp.zeros((pad, 128), jnp.int32)], axis=0)

    _, dperm2, kf2 = _make_sampling_kernel(E)(src2, dst2, dst, perm2)
    dperm = dperm2[:rows].reshape(E)
    keepf = kf2[:rows].reshape(E, 1)

    e_iota = jnp.arange(E, dtype=jnp.int32)
    Lsrc = jnp.zeros((_N_NODES,), jnp.int32).at[src].max(e_iota)
    Ldst = jnp.zeros((_N_NODES,), jnp.int32).at[dst].max(e_iota)

    A = jnp.take(h_src, Lsrc, axis=0) @ W1[:D]
    B = jnp.take(h_dst, Ldst, axis=0) @ W1[D:]

    G = _make_pair_gather_kernel(E)(A, B, src, dperm)

    w2row = W2.reshape(1, D)
    bias = jnp.concatenate(
        [b1.reshape(1, D), jnp.broadcast_to(b2.reshape(1, 1), (1, D))], axis=0)

    pos_sum, neg_sum, keep_sum = _fused_loss(h_src, h_dst, G, keepf, W1, w2row, bias)
    return -(pos_sum[0, 0] / E + neg_sum[0, 0] / keep_sum[0, 0])
